# Initial kernel scaffold; baseline (speedup 1.0000x reference)
import jax
import jax.numpy as jnp
from jax.experimental import pallas as pl


def kernel(x, adj, weight, bn_gamma, bn_beta, ln_gamma, ln_beta):
    raise NotImplementedError("write your pallas kernel here")


# speedup vs baseline: 1.3657x; 1.3657x over previous
import functools

import jax
import jax.numpy as jnp
from jax.experimental import pallas as pl
from jax.experimental.pallas import tpu as pltpu

_BN_EPS = 1e-5
_LN_EPS = 1e-5


def _matmul_stats_kernel(x_ref, adj_ref, w_ref, pre_ref, psum_ref, psumsq_ref):
    adj = adj_ref[0]
    x = x_ref[0]
    w = w_ref[...]

    deg = jnp.sum(adj.astype(jnp.float32), axis=-1, keepdims=True)
    d_inv_sqrt = jnp.where(deg > 0, jax.lax.rsqrt(deg), 0.0)

    xs = (x.astype(jnp.float32) * d_inv_sqrt).astype(jnp.bfloat16)
    t = jnp.dot(adj, xs, preferred_element_type=jnp.float32)
    h = (t * d_inv_sqrt).astype(jnp.bfloat16)
    out = jnp.dot(h, w, preferred_element_type=jnp.float32)

    pre_ref[0] = out.astype(jnp.bfloat16)
    psum_ref[0] = jnp.sum(out, axis=0, keepdims=True)
    psumsq_ref[0] = jnp.sum(out * out, axis=0, keepdims=True)


def _norm_kernel(inv_count, pre_ref, psum_ref, psumsq_ref,
                 bn_g_ref, bn_b_ref, ln_g_ref, ln_b_ref, o_ref):
    mean = jnp.sum(psum_ref[...], axis=0) * inv_count
    ex2 = jnp.sum(psumsq_ref[...], axis=0) * inv_count
    var = ex2 - mean * mean
    scale = bn_g_ref[...] * jax.lax.rsqrt(var + _BN_EPS)
    shift = bn_b_ref[...] - mean * scale

    bn = pre_ref[...].astype(jnp.float32) * scale + shift
    mu = jnp.mean(bn, axis=-1, keepdims=True)
    v = jnp.mean((bn - mu) ** 2, axis=-1, keepdims=True)
    ln = (bn - mu) * jax.lax.rsqrt(v + _LN_EPS)
    o_ref[...] = ln * ln_g_ref[...] + ln_b_ref[...]


def kernel(x, adj, weight, bn_gamma, bn_beta, ln_gamma, ln_beta):
    B, N, Fin = x.shape
    Fout = weight.shape[1]

    x_bf = x.astype(jnp.bfloat16)
    adj_bf = adj.astype(jnp.bfloat16)
    w_bf = weight.astype(jnp.bfloat16)

    parallel = pltpu.CompilerParams(dimension_semantics=("parallel",))

    pre, psum, psumsq = pl.pallas_call(
        _matmul_stats_kernel,
        grid=(B,),
        in_specs=[
            pl.BlockSpec((1, N, Fin), lambda b: (b, 0, 0)),
            pl.BlockSpec((1, N, N), lambda b: (b, 0, 0)),
            pl.BlockSpec((Fin, Fout), lambda b: (0, 0)),
        ],
        out_specs=[
            pl.BlockSpec((1, N, Fout), lambda b: (b, 0, 0)),
            pl.BlockSpec((1, 1, Fout), lambda b: (b, 0, 0)),
            pl.BlockSpec((1, 1, Fout), lambda b: (b, 0, 0)),
        ],
        out_shape=[
            jax.ShapeDtypeStruct((B, N, Fout), jnp.bfloat16),
            jax.ShapeDtypeStruct((B, 1, Fout), jnp.float32),
            jax.ShapeDtypeStruct((B, 1, Fout), jnp.float32),
        ],
        compiler_params=parallel,
    )(x_bf, adj_bf, w_bf)

    G = 8
    out = pl.pallas_call(
        functools.partial(_norm_kernel, 1.0 / float(B * N)),
        grid=(B // G,),
        in_specs=[
            pl.BlockSpec((G, N, Fout), lambda b: (b, 0, 0)),
            pl.BlockSpec((B, 1, Fout), lambda b: (0, 0, 0)),
            pl.BlockSpec((B, 1, Fout), lambda b: (0, 0, 0)),
            pl.BlockSpec((1, Fout), lambda b: (0, 0)),
            pl.BlockSpec((1, Fout), lambda b: (0, 0)),
            pl.BlockSpec((1, Fout), lambda b: (0, 0)),
            pl.BlockSpec((1, Fout), lambda b: (0, 0)),
        ],
        out_specs=pl.BlockSpec((G, N, Fout), lambda b: (b, 0, 0)),
        out_shape=jax.ShapeDtypeStruct((B, N, Fout), jnp.float32),
        compiler_params=parallel,
    )(pre, psum, psumsq, bn_gamma, bn_beta, ln_gamma, ln_beta)

    return out


# speedup vs baseline: 2.5728x; 1.8839x over previous
import functools

import jax
import jax.numpy as jnp
from jax.experimental import pallas as pl
from jax.experimental.pallas import tpu as pltpu

_BN_EPS = 1e-5
_LN_EPS = 1e-5


def _matmul_stats_kernel(g_sz, x_ref, adj_ref, w_ref,
                         pre_ref, psum_ref, psumsq_ref):
    adj = adj_ref[...]
    x = x_ref[...]
    w = w_ref[...]
    n = adj.shape[1]
    fin = x.shape[2]

    deg = jnp.sum(adj.astype(jnp.float32), axis=-1, keepdims=True)
    d_inv_sqrt = jnp.where(deg > 0, jax.lax.rsqrt(deg), 0.0)

    xs = (x.astype(jnp.float32) * d_inv_sqrt).astype(jnp.bfloat16)

    t = jnp.concatenate(
        [jnp.dot(adj[g], xs[g], preferred_element_type=jnp.float32)
         for g in range(g_sz)], axis=0)
    h = (t * d_inv_sqrt.reshape(g_sz * n, 1)).astype(jnp.bfloat16)
    out = jnp.dot(h, w, preferred_element_type=jnp.float32)

    pre_ref[...] = out.reshape(g_sz, n, -1).astype(jnp.bfloat16)
    psum_ref[...] = jnp.sum(out, axis=0, keepdims=True)[None]
    psumsq_ref[...] = jnp.sum(out * out, axis=0, keepdims=True)[None]


def _norm_kernel(inv_count, pre_ref, psum_ref, psumsq_ref,
                 bn_g_ref, bn_b_ref, ln_g_ref, ln_b_ref, o_ref):
    mean = jnp.sum(psum_ref[...], axis=0) * inv_count
    ex2 = jnp.sum(psumsq_ref[...], axis=0) * inv_count
    var = ex2 - mean * mean
    scale = bn_g_ref[...] * jax.lax.rsqrt(var + _BN_EPS)
    shift = bn_b_ref[...] - mean * scale

    bn = pre_ref[...].astype(jnp.float32) * scale + shift
    mu = jnp.mean(bn, axis=-1, keepdims=True)
    v = jnp.mean((bn - mu) ** 2, axis=-1, keepdims=True)
    ln = (bn - mu) * jax.lax.rsqrt(v + _LN_EPS)
    o_ref[...] = ln * ln_g_ref[...] + ln_b_ref[...]


def kernel(x, adj, weight, bn_gamma, bn_beta, ln_gamma, ln_beta):
    B, N, Fin = x.shape
    Fout = weight.shape[1]

    x_bf = x.astype(jnp.bfloat16)
    adj_bf = adj.astype(jnp.bfloat16)
    w_bf = weight.astype(jnp.bfloat16)

    parallel = pltpu.CompilerParams(dimension_semantics=("parallel",))

    G1 = 8
    S = B // G1
    pre, psum, psumsq = pl.pallas_call(
        functools.partial(_matmul_stats_kernel, G1),
        grid=(S,),
        in_specs=[
            pl.BlockSpec((G1, N, Fin), lambda b: (b, 0, 0)),
            pl.BlockSpec((G1, N, N), lambda b: (b, 0, 0)),
            pl.BlockSpec((Fin, Fout), lambda b: (0, 0)),
        ],
        out_specs=[
            pl.BlockSpec((G1, N, Fout), lambda b: (b, 0, 0)),
            pl.BlockSpec((1, 1, Fout), lambda b: (b, 0, 0)),
            pl.BlockSpec((1, 1, Fout), lambda b: (b, 0, 0)),
        ],
        out_shape=[
            jax.ShapeDtypeStruct((B, N, Fout), jnp.bfloat16),
            jax.ShapeDtypeStruct((S, 1, Fout), jnp.float32),
            jax.ShapeDtypeStruct((S, 1, Fout), jnp.float32),
        ],
        compiler_params=parallel,
    )(x_bf, adj_bf, w_bf)

    G2 = 8
    out = pl.pallas_call(
        functools.partial(_norm_kernel, 1.0 / float(B * N)),
        grid=(B // G2,),
        in_specs=[
            pl.BlockSpec((G2, N, Fout), lambda b: (b, 0, 0)),
            pl.BlockSpec((S, 1, Fout), lambda b: (0, 0, 0)),
            pl.BlockSpec((S, 1, Fout), lambda b: (0, 0, 0)),
            pl.BlockSpec((1, Fout), lambda b: (0, 0)),
            pl.BlockSpec((1, Fout), lambda b: (0, 0)),
            pl.BlockSpec((1, Fout), lambda b: (0, 0)),
            pl.BlockSpec((1, Fout), lambda b: (0, 0)),
        ],
        out_specs=pl.BlockSpec((G2, N, Fout), lambda b: (b, 0, 0)),
        out_shape=jax.ShapeDtypeStruct((B, N, Fout), jnp.float32),
        compiler_params=parallel,
    )(pre, psum, psumsq, bn_gamma, bn_beta, ln_gamma, ln_beta)

    return out
